# baseline (speedup 1.0000x reference)
import functools

import jax
import jax.numpy as jnp
from jax import lax
from jax.experimental import pallas as pl
from jax.experimental.pallas import tpu as pltpu
from jax.experimental.pallas import tpu_sc as plsc

_MASK_RATIO = 0.5
_MASK_VALUE = 0.0


def _threefry_sortkeys(n):
    rotations = ((13, 15, 26, 6), (17, 29, 16, 24))
    k = (jnp.uint32(0), jnp.uint32(42),
         jnp.uint32(0) ^ jnp.uint32(42) ^ jnp.uint32(0x1BD11BDA))
    x0 = jnp.zeros_like(n) + k[0]
    x1 = n + k[1]
    for i in range(5):
        for r in rotations[i % 2]:
            x0 = x0 + x1
            x1 = (x1 << jnp.uint32(r)) | (x1 >> jnp.uint32(32 - r))
            x1 = x0 ^ x1
        x0 = x0 + k[(i + 1) % 3]
        x1 = x1 + k[(i + 2) % 3] + jnp.uint32(i + 1)
    bits = x0 ^ x1
    return (bits >> jnp.uint32(9)).astype(jnp.int32)


def _mask_body(x_ref, xm_ref, keys_scr, keep_scr):
    _, L, D, bs = x_ref.shape
    len_keep = int(L * (1 - _MASK_RATIO))

    @pl.when(pl.program_id(0) == 0)
    def _init():
        li = jax.lax.broadcasted_iota(jnp.int32, (L, bs), 0)
        bi = jax.lax.broadcasted_iota(jnp.int32, (L, bs), 1)
        keys = _threefry_sortkeys((bi * L + li).astype(jnp.uint32))
        keys_scr[...] = keys

        def body(j, rank):
            kj = keys_scr[pl.ds(j, 1), :]
            return rank + ((kj < keys) |
                           ((kj == keys) & (li > j))).astype(jnp.int32)

        rank = jax.lax.fori_loop(0, L, body, jnp.zeros((L, bs), jnp.int32))
        keep_scr[...] = (rank < len_keep).astype(jnp.float32)

    xm_ref[...] = x_ref[...] * keep_scr[...][None, :, None, :]


def _sc_mask(nv, L, bs):
    NC, NS, NL = 2, 16, 16
    NBG, NVG = 8, 4
    bw = bs // NBG
    nch = bw // NL
    mesh = plsc.VectorSubcoreMesh(core_axis_name="c", subcore_axis_name="s")

    @functools.partial(
        pl.kernel, mesh=mesh,
        out_type=jax.ShapeDtypeStruct((nv, L, bs), jnp.float32),
        scratch_types=[pltpu.VMEM((L * bw,), jnp.int32),
                       pltpu.VMEM((L, bw), jnp.float32)],
    )
    def k(out_hbm, keys_v, mask_v):
        wid = lax.axis_index("s") * NC + lax.axis_index("c")
        bg = wid % NBG
        vg = wid // NBG
        b0 = bg * bw
        lane = lax.iota(jnp.int32, NL)

        def keyloop(i, carry):
            l = i // nch
            c = i % nch
            n = ((b0 + c * NL + lane) * L + l).astype(jnp.uint32)
            keys_v[pl.ds(i * NL, NL)] = _threefry_sortkeys(n)
            return carry

        lax.fori_loop(0, L * nch, keyloop, 0)

        for c in range(nch):
            for g in range(0, L, NL):
                kls = [keys_v[pl.ds(((g + t) * nch + c) * NL, NL)]
                       for t in range(NL)]

                def jloop(j, accs, _kls=kls, _c=c):
                    kj = keys_v[pl.ds((j * nch + _c) * NL, NL)]
                    one = jnp.ones((NL,), jnp.int32)
                    zero = jnp.zeros((NL,), jnp.int32)
                    return tuple(a + jnp.where(kj < kl, one, zero)
                                 for a, kl in zip(accs, _kls))

                accs = lax.fori_loop(
                    0, L, jloop,
                    tuple(jnp.zeros((NL,), jnp.int32) for _ in range(NL)))
                for t in range(NL):
                    mask_v[g + t, pl.ds(c * NL, NL)] = jnp.where(
                        accs[t] >= L // 2,
                        jnp.full((NL,), 1.0, jnp.float32),
                        jnp.full((NL,), 0.0, jnp.float32))

        for vv in range(nv // NVG):
            pltpu.sync_copy(mask_v,
                            out_hbm.at[vg * (nv // NVG) + vv, :,
                                       pl.ds(b0, bw)])

    return k


def kernel(x):
    bs, nv, L, D = x.shape
    xt = jnp.transpose(x, (1, 2, 3, 0))
    mask_t = _sc_mask(nv, L, bs)()
    V = 2
    xm_t = pl.pallas_call(
        _mask_body,
        grid=(nv // V,),
        in_specs=[pl.BlockSpec((V, L, D, bs), lambda i: (i, 0, 0, 0))],
        out_specs=pl.BlockSpec((V, L, D, bs), lambda i: (i, 0, 0, 0)),
        out_shape=jax.ShapeDtypeStruct((nv, L, D, bs), jnp.float32),
        scratch_shapes=[pltpu.VMEM((L, bs), jnp.int32),
                        pltpu.VMEM((L, bs), jnp.float32)],
        compiler_params=pltpu.CompilerParams(
            dimension_semantics=("arbitrary",)),
    )(xt)
    return jnp.transpose(xm_t, (3, 0, 1, 2)), jnp.transpose(mask_t, (2, 0, 1))


# speedup vs baseline: 1.1708x; 1.1708x over previous
import jax
import jax.numpy as jnp
from jax.experimental import pallas as pl
from jax.experimental.pallas import tpu as pltpu

_MASK_RATIO = 0.5
_MASK_VALUE = 0.0


def _threefry_sortkeys(n):
    rotations = ((13, 15, 26, 6), (17, 29, 16, 24))
    k = (jnp.uint32(0), jnp.uint32(42),
         jnp.uint32(0) ^ jnp.uint32(42) ^ jnp.uint32(0x1BD11BDA))
    x0 = jnp.zeros_like(n) + k[0]
    x1 = n + k[1]
    for i in range(5):
        for r in rotations[i % 2]:
            x0 = x0 + x1
            x1 = (x1 << jnp.uint32(r)) | (x1 >> jnp.uint32(32 - r))
            x1 = x0 ^ x1
        x0 = x0 + k[(i + 1) % 3]
        x1 = x1 + k[(i + 2) % 3] + jnp.uint32(i + 1)
    bits = x0 ^ x1
    return (bits >> jnp.uint32(9)).astype(jnp.int32)


def _mask_body(x_ref, xm_ref, mask_ref, keys_scr, keep_scr, maskv_scr):
    _, L, D, bs = x_ref.shape
    len_keep = int(L * (1 - _MASK_RATIO))

    @pl.when(pl.program_id(0) == 0)
    def _init():
        li = jax.lax.broadcasted_iota(jnp.int32, (L, bs), 0)
        bi = jax.lax.broadcasted_iota(jnp.int32, (L, bs), 1)
        keys = _threefry_sortkeys((bi * L + li).astype(jnp.uint32))
        keys_scr[...] = keys

        def body(j, rank):
            kj = keys_scr[pl.ds(j, 1), :]
            return rank + ((kj < keys) |
                           ((kj == keys) & (li > j))).astype(jnp.int32)

        rank = jax.lax.fori_loop(0, L, body, jnp.zeros((L, bs), jnp.int32))
        maskv = (rank >= len_keep).astype(jnp.float32)
        maskv_scr[...] = maskv
        keep_scr[...] = 1.0 - maskv

    @pl.when(pl.program_id(0) == 0)
    def _write_mask():
        mask_ref[...] = jnp.broadcast_to(maskv_scr[...][None, :, :],
                                         mask_ref.shape)

    xm_ref[...] = x_ref[...] * keep_scr[...][None, :, None, :]


def kernel(x):
    bs, nv, L, D = x.shape
    xt = jnp.transpose(x, (1, 2, 3, 0))
    V = 2
    xm_t, mask_t = pl.pallas_call(
        _mask_body,
        grid=(nv // V,),
        in_specs=[pl.BlockSpec((V, L, D, bs), lambda i: (i, 0, 0, 0))],
        out_specs=[pl.BlockSpec((V, L, D, bs), lambda i: (i, 0, 0, 0)),
                   pl.BlockSpec((nv, L, bs), lambda i: (0, 0, 0))],
        out_shape=[jax.ShapeDtypeStruct((nv, L, D, bs), jnp.float32),
                   jax.ShapeDtypeStruct((nv, L, bs), jnp.float32)],
        scratch_shapes=[pltpu.VMEM((L, bs), jnp.int32),
                        pltpu.VMEM((L, bs), jnp.float32),
                        pltpu.VMEM((L, bs), jnp.float32)],
        compiler_params=pltpu.CompilerParams(
            dimension_semantics=("arbitrary",)),
    )(xt)
    return jnp.transpose(xm_t, (3, 0, 1, 2)), jnp.transpose(mask_t, (2, 0, 1))
